# baseline (speedup 1.0000x reference)
import functools

import jax
import jax.numpy as jnp
from jax.experimental import pallas as pl
from jax.experimental.pallas import tpu as pltpu

E = 64


def _gelu_exact(x):
    return 0.5 * x * (1.0 + jax.lax.erf(x * 0.7071067811865476))


def _layer_norm(x, g, b, eps=1e-5):
    mu = jnp.mean(x, axis=-1, keepdims=True)
    var = jnp.mean((x - mu) ** 2, axis=-1, keepdims=True)
    return (x - mu) * jax.lax.rsqrt(var + eps) * g + b


def _dense_s(idx, w, rows):
    k_dim = idx.shape[-1]
    iota = jax.lax.broadcasted_iota(jnp.int32, (rows, E), 1)
    s = jnp.zeros((rows, E), jnp.float32)
    for k in range(k_dim):
        s = s + jnp.where(idx[:, k : k + 1] == iota, w[:, k : k + 1], 0.0)
    return s


def _fused_kernel(
    idx_ref, w_ref, x_hbm, we_ref, be_ref, ge_ref, bbe_ref,
    wn_ref, bn_ref, gn_ref, bbn_ref, o_hbm,
    x_vmem, s_vmem, he_vmem, in_sems, out_sems, *, ch, n_chunks, n_b,
):
    b = pl.program_id(0)
    p = jax.lax.rem(b, 2)

    def in_copy(src_b, buf, c):
        return pltpu.make_async_copy(
            x_hbm.at[src_b, pl.ds(c * ch, ch), :],
            x_vmem.at[buf, pl.ds(c * ch, ch), :],
            in_sems.at[buf, c],
        )

    def out_copy(buf, c):
        return pltpu.make_async_copy(
            x_vmem.at[buf, pl.ds(c * ch, ch), :],
            o_hbm.at[b, pl.ds(c * ch, ch), :],
            out_sems.at[buf, c],
        )

    @pl.when(b == 0)
    def _():
        for c in range(n_chunks):
            in_copy(0, 0, c).start()

    for c in range(n_chunks):
        lo, hi = c * ch, (c + 1) * ch
        s = _dense_s(idx_ref[0, lo:hi], w_ref[0, lo:hi], ch)
        s_vmem[lo:hi, :] = s
        in_copy(b, p, c).wait()
        acc = jax.lax.dot_general(
            s, x_vmem[p, lo:hi, :], (((0,), (0,)), ((), ())),
            preferred_element_type=jnp.float32,
        )
        if c == 0:
            he_vmem[...] = acc
        else:
            he_vmem[...] = he_vmem[...] + acc

        @pl.when(b >= 1)
        def _():
            out_copy(1 - p, c).wait()

        @pl.when(b + 1 < n_b)
        def _():
            in_copy(b + 1, 1 - p, c).start()

    h = jnp.dot(
        he_vmem[...].astype(jnp.bfloat16), we_ref[...],
        preferred_element_type=jnp.float32,
    )
    hep = _layer_norm(_gelu_exact(h + be_ref[...]), ge_ref[...], bbe_ref[...])

    for c in range(n_chunks):
        lo, hi = c * ch, (c + 1) * ch
        y = jnp.dot(s_vmem[lo:hi, :], hep, preferred_element_type=jnp.float32)
        z = jnp.dot(
            y.astype(jnp.bfloat16), wn_ref[...],
            preferred_element_type=jnp.float32,
        )
        z = _layer_norm(_gelu_exact(z + bn_ref[...]), gn_ref[...], bbn_ref[...])
        x_vmem[p, lo:hi, :] = z + x_vmem[p, lo:hi, :]
        out_copy(p, c).start()

    @pl.when(b == n_b - 1)
    def _():
        for c in range(n_chunks):
            out_copy(p, c).wait()


def kernel(X, edge_idx, edge_w, We, be, ge, bbe, Wn, bn, gn, bbn):
    B, N, D = X.shape
    K = edge_idx.shape[-1]
    ch = min(512, N)
    n_chunks = N // ch

    idx = edge_idx.astype(jnp.int32)
    w = edge_w.astype(jnp.float32)
    web = We.astype(jnp.bfloat16)
    wnb = Wn.astype(jnp.bfloat16)
    be2, ge2, bbe2 = be.reshape(1, D), ge.reshape(1, D), bbe.reshape(1, D)
    bn2, gn2, bbn2 = bn.reshape(1, D), gn.reshape(1, D), bbn.reshape(1, D)

    blk_nk = pl.BlockSpec((1, N, K), lambda b: (b, 0, 0))
    blk_dd = pl.BlockSpec((D, D), lambda b: (0, 0))
    blk_1d = pl.BlockSpec((1, D), lambda b: (0, 0))
    blk_any = pl.BlockSpec(memory_space=pl.ANY)

    out = pl.pallas_call(
        functools.partial(_fused_kernel, ch=ch, n_chunks=n_chunks, n_b=B),
        grid=(B,),
        in_specs=[blk_nk, blk_nk, blk_any, blk_dd, blk_1d, blk_1d, blk_1d,
                  blk_dd, blk_1d, blk_1d, blk_1d],
        out_specs=blk_any,
        out_shape=jax.ShapeDtypeStruct((B, N, D), jnp.float32),
        scratch_shapes=[
            pltpu.VMEM((2, N, D), jnp.float32),
            pltpu.VMEM((N, E), jnp.float32),
            pltpu.VMEM((E, D), jnp.float32),
            pltpu.SemaphoreType.DMA((2, n_chunks)),
            pltpu.SemaphoreType.DMA((2, n_chunks)),
        ],
        compiler_params=pltpu.CompilerParams(
            dimension_semantics=("arbitrary",)
        ),
    )(idx, w, X, web, be2, ge2, bbe2, wnb, bn2, gn2, bbn2)
    return out


# speedup vs baseline: 1.0217x; 1.0217x over previous
import functools

import jax
import jax.numpy as jnp
from jax.experimental import pallas as pl
from jax.experimental.pallas import tpu as pltpu

E = 64


def _gelu_exact(x):
    return 0.5 * x * (1.0 + jax.lax.erf(x * 0.7071067811865476))


def _layer_norm(x, g, b, eps=1e-5):
    mu = jnp.mean(x, axis=-1, keepdims=True)
    var = jnp.mean((x - mu) ** 2, axis=-1, keepdims=True)
    return (x - mu) * jax.lax.rsqrt(var + eps) * g + b


def _dense_s(idx, w, rows):
    k_dim = idx.shape[-1]
    iota = jax.lax.broadcasted_iota(jnp.int32, (rows, E), 1)
    s = jnp.zeros((rows, E), jnp.float32)
    for k in range(k_dim):
        s = s + jnp.where(idx[:, k : k + 1] == iota, w[:, k : k + 1], 0.0)
    return s


def _fused_kernel(
    idx_ref, w_ref, x_hbm, we_ref, be_ref, ge_ref, bbe_ref,
    wn_ref, bn_ref, gn_ref, bbn_ref, o_hbm,
    x_vmem, s_vmem, he_vmem, in_sems, out_sems, *, ch, n_chunks, n_b,
):
    b = pl.program_id(0)
    p = jax.lax.rem(b, 2)

    def in_copy(src_b, buf, c):
        return pltpu.make_async_copy(
            x_hbm.at[src_b, pl.ds(c * ch, ch), :],
            x_vmem.at[buf, pl.ds(c * ch, ch), :],
            in_sems.at[buf, c],
        )

    def out_copy(buf, c):
        return pltpu.make_async_copy(
            x_vmem.at[buf, pl.ds(c * ch, ch), :],
            o_hbm.at[b, pl.ds(c * ch, ch), :],
            out_sems.at[buf, c],
        )

    @pl.when(b == 0)
    def _():
        for c in range(n_chunks):
            in_copy(0, 0, c).start()

    for c in range(n_chunks):
        lo, hi = c * ch, (c + 1) * ch
        s = _dense_s(idx_ref[0, lo:hi], w_ref[0, lo:hi], ch)
        s_vmem[lo:hi, :] = s
        in_copy(b, p, c).wait()
        acc = jax.lax.dot_general(
            s, x_vmem[p, lo:hi, :], (((0,), (0,)), ((), ())),
            preferred_element_type=jnp.float32,
        )
        if c == 0:
            he_vmem[...] = acc
        else:
            he_vmem[...] = he_vmem[...] + acc

    @pl.when(b >= 1)
    def _():
        for c in range(n_chunks):
            out_copy(1 - p, c).wait()

    @pl.when(b + 1 < n_b)
    def _():
        for c in range(n_chunks):
            in_copy(b + 1, 1 - p, c).start()

    h = jnp.dot(
        he_vmem[...].astype(jnp.bfloat16), we_ref[...],
        preferred_element_type=jnp.float32,
    )
    hep = _layer_norm(_gelu_exact(h + be_ref[...]), ge_ref[...], bbe_ref[...])

    for c in range(n_chunks):
        lo, hi = c * ch, (c + 1) * ch
        y = jnp.dot(s_vmem[lo:hi, :], hep, preferred_element_type=jnp.float32)
        z = jnp.dot(
            y.astype(jnp.bfloat16), wn_ref[...],
            preferred_element_type=jnp.float32,
        )
        z = _layer_norm(_gelu_exact(z + bn_ref[...]), gn_ref[...], bbn_ref[...])
        x_vmem[p, lo:hi, :] = z + x_vmem[p, lo:hi, :]
        out_copy(p, c).start()

    @pl.when(b == n_b - 1)
    def _():
        for c in range(n_chunks):
            out_copy(p, c).wait()


def kernel(X, edge_idx, edge_w, We, be, ge, bbe, Wn, bn, gn, bbn):
    B, N, D = X.shape
    K = edge_idx.shape[-1]
    ch = min(512, N)
    n_chunks = N // ch

    idx = edge_idx.astype(jnp.int32)
    w = edge_w.astype(jnp.float32)
    web = We.astype(jnp.bfloat16)
    wnb = Wn.astype(jnp.bfloat16)
    be2, ge2, bbe2 = be.reshape(1, D), ge.reshape(1, D), bbe.reshape(1, D)
    bn2, gn2, bbn2 = bn.reshape(1, D), gn.reshape(1, D), bbn.reshape(1, D)

    blk_nk = pl.BlockSpec((1, N, K), lambda b: (b, 0, 0))
    blk_dd = pl.BlockSpec((D, D), lambda b: (0, 0))
    blk_1d = pl.BlockSpec((1, D), lambda b: (0, 0))
    blk_any = pl.BlockSpec(memory_space=pl.ANY)

    out = pl.pallas_call(
        functools.partial(_fused_kernel, ch=ch, n_chunks=n_chunks, n_b=B),
        grid=(B,),
        in_specs=[blk_nk, blk_nk, blk_any, blk_dd, blk_1d, blk_1d, blk_1d,
                  blk_dd, blk_1d, blk_1d, blk_1d],
        out_specs=blk_any,
        out_shape=jax.ShapeDtypeStruct((B, N, D), jnp.float32),
        scratch_shapes=[
            pltpu.VMEM((2, N, D), jnp.float32),
            pltpu.VMEM((N, E), jnp.float32),
            pltpu.VMEM((E, D), jnp.float32),
            pltpu.SemaphoreType.DMA((2, n_chunks)),
            pltpu.SemaphoreType.DMA((2, n_chunks)),
        ],
        compiler_params=pltpu.CompilerParams(
            dimension_semantics=("arbitrary",)
        ),
    )(idx, w, X, web, be2, ge2, bbe2, wnb, bn2, gn2, bbn2)
    return out


# speedup vs baseline: 1.0612x; 1.0386x over previous
import functools

import jax
import jax.numpy as jnp
from jax.experimental import pallas as pl
from jax.experimental.pallas import tpu as pltpu

E = 64


def _gelu_exact(x):
    return 0.5 * x * (1.0 + jax.lax.erf(x * 0.7071067811865476))


def _layer_norm(x, g, b, eps=1e-5):
    mu = jnp.mean(x, axis=-1, keepdims=True)
    var = jnp.mean((x - mu) ** 2, axis=-1, keepdims=True)
    return (x - mu) * jax.lax.rsqrt(var + eps) * g + b


def _dense_s(idx, w, rows):
    k_dim = idx.shape[-1]
    iota = jax.lax.broadcasted_iota(jnp.int32, (rows, E), 1)
    s = jnp.zeros((rows, E), jnp.float32)
    for k in range(k_dim):
        s = s + jnp.where(idx[:, k : k + 1] == iota, w[:, k : k + 1], 0.0)
    return s


def _fused_kernel(
    idx_ref, w_ref, x_hbm, we_ref, be_ref, ge_ref, bbe_ref,
    wn_ref, bn_ref, gn_ref, bbn_ref, o_hbm,
    x_vmem, s_vmem, he_vmem, in_sems, out_sems, *, ch, n_chunks, n_b,
):
    b = pl.program_id(0)
    p = jax.lax.rem(b, 2)

    def in_copy(src_b, buf, c):
        return pltpu.make_async_copy(
            x_hbm.at[src_b, pl.ds(c * ch, ch), :],
            x_vmem.at[buf, pl.ds(c * ch, ch), :],
            in_sems.at[buf, c],
        )

    def out_copy(buf, c):
        return pltpu.make_async_copy(
            x_vmem.at[buf, pl.ds(c * ch, ch), :],
            o_hbm.at[b, pl.ds(c * ch, ch), :],
            out_sems.at[buf, c],
        )

    @pl.when(b == 0)
    def _():
        for c in range(n_chunks):
            in_copy(0, 0, c).start()

    for c in range(n_chunks):
        lo, hi = c * ch, (c + 1) * ch
        s = _dense_s(idx_ref[0, lo:hi], w_ref[0, lo:hi], ch)
        s_vmem[lo:hi, :] = s
        in_copy(b, p, c).wait()
        acc = jax.lax.dot_general(
            s, x_vmem[p, lo:hi, :], (((0,), (0,)), ((), ())),
            preferred_element_type=jnp.float32,
        )
        if c == 0:
            he_vmem[...] = acc
        else:
            he_vmem[...] = he_vmem[...] + acc

    @pl.when(b >= 1)
    def _():
        for c in range(n_chunks):
            out_copy(1 - p, c).wait()

    @pl.when(b + 1 < n_b)
    def _():
        for c in range(n_chunks):
            in_copy(b + 1, 1 - p, c).start()

    h = jnp.dot(he_vmem[...], we_ref[...], preferred_element_type=jnp.float32)
    hep = _layer_norm(_gelu_exact(h + be_ref[...]), ge_ref[...], bbe_ref[...])

    for c in range(n_chunks):
        lo, hi = c * ch, (c + 1) * ch
        y = jnp.dot(s_vmem[lo:hi, :], hep, preferred_element_type=jnp.float32)
        z = jnp.dot(y, wn_ref[...], preferred_element_type=jnp.float32)
        z = _layer_norm(_gelu_exact(z + bn_ref[...]), gn_ref[...], bbn_ref[...])
        x_vmem[p, lo:hi, :] = z + x_vmem[p, lo:hi, :]
        out_copy(p, c).start()

    @pl.when(b == n_b - 1)
    def _():
        for c in range(n_chunks):
            out_copy(p, c).wait()


def kernel(X, edge_idx, edge_w, We, be, ge, bbe, Wn, bn, gn, bbn):
    B, N, D = X.shape
    K = edge_idx.shape[-1]
    ch = min(1024, N)
    n_chunks = N // ch

    idx = edge_idx.astype(jnp.int32)
    w = edge_w.astype(jnp.float32)
    be2, ge2, bbe2 = be.reshape(1, D), ge.reshape(1, D), bbe.reshape(1, D)
    bn2, gn2, bbn2 = bn.reshape(1, D), gn.reshape(1, D), bbn.reshape(1, D)

    blk_nk = pl.BlockSpec((1, N, K), lambda b: (b, 0, 0))
    blk_dd = pl.BlockSpec((D, D), lambda b: (0, 0))
    blk_1d = pl.BlockSpec((1, D), lambda b: (0, 0))
    blk_any = pl.BlockSpec(memory_space=pl.ANY)

    out = pl.pallas_call(
        functools.partial(_fused_kernel, ch=ch, n_chunks=n_chunks, n_b=B),
        grid=(B,),
        in_specs=[blk_nk, blk_nk, blk_any, blk_dd, blk_1d, blk_1d, blk_1d,
                  blk_dd, blk_1d, blk_1d, blk_1d],
        out_specs=blk_any,
        out_shape=jax.ShapeDtypeStruct((B, N, D), jnp.float32),
        scratch_shapes=[
            pltpu.VMEM((2, N, D), jnp.float32),
            pltpu.VMEM((N, E), jnp.float32),
            pltpu.VMEM((E, D), jnp.float32),
            pltpu.SemaphoreType.DMA((2, n_chunks)),
            pltpu.SemaphoreType.DMA((2, n_chunks)),
        ],
        compiler_params=pltpu.CompilerParams(
            dimension_semantics=("arbitrary",)
        ),
    )(idx, w, X, We, be2, ge2, bbe2, Wn, bn2, gn2, bbn2)
    return out
